# Initial kernel scaffold; baseline (speedup 1.0000x reference)
import jax
import jax.numpy as jnp
from jax.experimental import pallas as pl


def kernel(x, coords, edge_attr, bc_disp, bc_rot, params, edge_index):
    raise NotImplementedError("write your pallas kernel here")


# speedup vs baseline: 3.3773x; 3.3773x over previous
import functools

import jax
import jax.numpy as jnp
from jax import lax
from jax.experimental import pallas as pl
from jax.experimental.pallas import tpu as pltpu
from jax.experimental.pallas import tpu_sc as plsc

_H = 128
_NLAYERS = 6
_NC = 2
_NS = 16
_LAN = 16
_CH = 80
_BLK_N = 2000
_BLK_E = 2000


def _mm(a, b):
    return jnp.dot(a, b, preferred_element_type=jnp.float32)


def _wprep_body(U2, c2, We, Wcomb, ccomb):
    for l in range(_NLAYERS):
        Wcomb[l] = _mm(U2[...], We[l])
        ccomb[l] = _mm(c2[...], We[l])


def _wprep(U2, c2, We):
    return pl.pallas_call(
        _wprep_body,
        out_shape=[
            jax.ShapeDtypeStruct((_NLAYERS, _H, _H), jnp.float32),
            jax.ShapeDtypeStruct((_NLAYERS, 1, _H), jnp.float32),
        ],
    )(U2, c2, We)


def _edge_enc_body(ea, U1, c1, Wcomb, ccomb, out):
    ehid = jnp.maximum(_mm(ea[...], U1[...]) + c1[...], 0.0)
    for l in range(_NLAYERS):
        out[l] = _mm(ehid, Wcomb[l]) + ccomb[l]


def _edge_enc(ea, U1, c1, Wcomb, ccomb):
    E = ea.shape[0]
    grid = E // _BLK_E
    return pl.pallas_call(
        _edge_enc_body,
        grid=(grid,),
        in_specs=[
            pl.BlockSpec((_BLK_E, ea.shape[1]), lambda i: (i, 0)),
            pl.BlockSpec(U1.shape, lambda i: (0, 0)),
            pl.BlockSpec(c1.shape, lambda i: (0, 0)),
            pl.BlockSpec(Wcomb.shape, lambda i: (0, 0, 0)),
            pl.BlockSpec(ccomb.shape, lambda i: (0, 0, 0)),
        ],
        out_specs=pl.BlockSpec((_NLAYERS, _BLK_E, _H), lambda i: (0, i, 0)),
        out_shape=jax.ShapeDtypeStruct((_NLAYERS, E, _H), jnp.float32),
    )(ea, U1, c1, Wcomb, ccomb)


def _node_enc_body(x6, P1, q1, P2, q2, out):
    t = jnp.maximum(_mm(x6[...], P1[...]) + q1[...], 0.0)
    out[...] = _mm(t, P2[...]) + q2[...]


def _node_enc(x6, P1, q1, P2, q2):
    N = x6.shape[0]
    return pl.pallas_call(
        _node_enc_body,
        grid=(N // _BLK_N,),
        in_specs=[
            pl.BlockSpec((_BLK_N, x6.shape[1]), lambda i: (i, 0)),
            pl.BlockSpec(P1.shape, lambda i: (0, 0)),
            pl.BlockSpec(q1.shape, lambda i: (0, 0)),
            pl.BlockSpec(P2.shape, lambda i: (0, 0)),
            pl.BlockSpec(q2.shape, lambda i: (0, 0)),
        ],
        out_specs=pl.BlockSpec((_BLK_N, _H), lambda i: (i, 0)),
        out_shape=jax.ShapeDtypeStruct((N, _H), jnp.float32),
    )(x6, P1, q1, P2, q2)


def _ab_body(h, Ws, Wd, b1, A, B):
    hv = h[...]
    A[...] = _mm(hv, Ws[...])
    B[...] = _mm(hv, Wd[...]) + b1[...]


def _ab(h, Ws, Wd, b1):
    N = h.shape[0]
    return pl.pallas_call(
        _ab_body,
        grid=(N // _BLK_N,),
        in_specs=[
            pl.BlockSpec((_BLK_N, _H), lambda i: (i, 0)),
            pl.BlockSpec(Ws.shape, lambda i: (0, 0)),
            pl.BlockSpec(Wd.shape, lambda i: (0, 0)),
            pl.BlockSpec(b1.shape, lambda i: (0, 0)),
        ],
        out_specs=[
            pl.BlockSpec((_BLK_N, _H), lambda i: (i, 0)),
            pl.BlockSpec((_BLK_N, _H), lambda i: (i, 0)),
        ],
        out_shape=[
            jax.ShapeDtypeStruct((N, _H), jnp.float32),
            jax.ShapeDtypeStruct((N, _H), jnp.float32),
        ],
    )(h, Ws, Wd, b1)


def _node_upd_body(S2, deg, h, W2, b2, V1h, V1a, d1, V2, d2, out):
    S = S2[0] + S2[1]
    dg = deg[0, :, 0:1] + deg[1, :, 0:1]
    agg = _mm(S, W2[...]) + dg * b2[...]
    hv = h[...]
    t = jnp.maximum(_mm(hv, V1h[...]) + _mm(agg, V1a[...]) + d1[...], 0.0)
    out[...] = hv + _mm(t, V2[...]) + d2[...]


def _node_upd(S2, deg, h, W2, b2, V1h, V1a, d1, V2, d2):
    N = h.shape[0]
    return pl.pallas_call(
        _node_upd_body,
        grid=(N // _BLK_N,),
        in_specs=[
            pl.BlockSpec((_NC, _BLK_N, _H), lambda i: (0, i, 0)),
            pl.BlockSpec((_NC, _BLK_N, _LAN), lambda i: (0, i, 0)),
            pl.BlockSpec((_BLK_N, _H), lambda i: (i, 0)),
            pl.BlockSpec(W2.shape, lambda i: (0, 0)),
            pl.BlockSpec(b2.shape, lambda i: (0, 0)),
            pl.BlockSpec(V1h.shape, lambda i: (0, 0)),
            pl.BlockSpec(V1a.shape, lambda i: (0, 0)),
            pl.BlockSpec(d1.shape, lambda i: (0, 0)),
            pl.BlockSpec(V2.shape, lambda i: (0, 0)),
            pl.BlockSpec(d2.shape, lambda i: (0, 0)),
        ],
        out_specs=pl.BlockSpec((_BLK_N, _H), lambda i: (i, 0)),
        out_shape=jax.ShapeDtypeStruct((N, _H), jnp.float32),
    )(S2, deg, h, W2, b2, V1h, V1a, d1, V2, d2)


def _dec_body(h, c2d, bcd, bcr, D1h, D1c, e1, D2, e2, D3, e3, out):
    t = jnp.maximum(_mm(h[...], D1h[...]) + _mm(c2d[...], D1c[...]) + e1[...], 0.0)
    t = jnp.maximum(_mm(t, D2[...]) + e2[...], 0.0)
    p = _mm(t, D3[...]) + e3[...]
    col = lax.broadcasted_iota(jnp.int32, p.shape, 1)
    mask = jnp.where(col < 2, 1.0 - bcd[...], 1.0 - bcr[...])
    out[...] = p * mask


def _dec(h, c2d, bcd, bcr, D1h, D1c, e1, D2, e2, D3, e3):
    N = h.shape[0]
    return pl.pallas_call(
        _dec_body,
        grid=(N // _BLK_N,),
        in_specs=[
            pl.BlockSpec((_BLK_N, _H), lambda i: (i, 0)),
            pl.BlockSpec((_BLK_N, 2), lambda i: (i, 0)),
            pl.BlockSpec((_BLK_N, 1), lambda i: (i, 0)),
            pl.BlockSpec((_BLK_N, 1), lambda i: (i, 0)),
            pl.BlockSpec(D1h.shape, lambda i: (0, 0)),
            pl.BlockSpec(D1c.shape, lambda i: (0, 0)),
            pl.BlockSpec(e1.shape, lambda i: (0, 0)),
            pl.BlockSpec(D2.shape, lambda i: (0, 0)),
            pl.BlockSpec(e2.shape, lambda i: (0, 0)),
            pl.BlockSpec(D3.shape, lambda i: (0, 0)),
            pl.BlockSpec(e3.shape, lambda i: (0, 0)),
        ],
        out_specs=pl.BlockSpec((_BLK_N, 3), lambda i: (i, 0)),
        out_shape=jax.ShapeDtypeStruct((N, 3), jnp.float32),
    )(h, c2d, bcd, bcr, D1h, D1c, e1, D2, e2, D3, e3)


def _edge_pass(A, B, C_all, src, dst, zeros, layer):
    N = A.shape[0]
    E = src.shape[0]
    per_w = E // (_NC * _NS)
    n_chunks = per_w // _CH
    rows_pt = (N // _NS) // 8 * 8
    tail = N - rows_pt * _NS
    mesh = plsc.VectorSubcoreMesh(core_axis_name="c", subcore_axis_name="s",
                                  num_cores=_NC, num_subcores=_NS)

    @functools.partial(
        pl.kernel,
        out_type=jax.ShapeDtypeStruct((_NC, N, _H), jnp.float32),
        mesh=mesh,
        scratch_types=[
            pltpu.VMEM((_CH,), jnp.int32),
            pltpu.VMEM((_CH,), jnp.int32),
            pltpu.VMEM((_CH, _H), jnp.float32),
            pltpu.VMEM((_CH, _H), jnp.float32),
            pltpu.VMEM((_CH, _H), jnp.float32),
            pltpu.VMEM_SHARED((N, _H), jnp.float32),
            pltpu.SemaphoreType.DMA,
            pltpu.SemaphoreType.DMA,
        ],
    )
    def k(A_h, B_h, C_h, src_h, dst_h, z_h, out_h, isv, idv, av, bv, cv,
          S_sh, sa, sb):
        cid = lax.axis_index("c")
        sid = lax.axis_index("s")
        r0 = sid * rows_pt
        pltpu.sync_copy(z_h.at[pl.ds(r0, rows_pt)], S_sh.at[pl.ds(r0, rows_pt)])
        if tail:
            @pl.when(sid == _NS - 1)
            def _zero_tail():
                pltpu.sync_copy(z_h.at[pl.ds(rows_pt * _NS, tail)],
                                S_sh.at[pl.ds(rows_pt * _NS, tail)])
        plsc.subcore_barrier()
        base = (cid * _NS + sid) * per_w

        def chunk(i, carry):
            off = base + i * _CH
            pltpu.sync_copy(src_h.at[pl.ds(off, _CH)], isv)
            pltpu.sync_copy(dst_h.at[pl.ds(off, _CH)], idv)
            ca = pltpu.async_copy(A_h.at[isv], av, sa)
            cb = pltpu.async_copy(B_h.at[idv], bv, sb)
            pltpu.sync_copy(C_h.at[layer, pl.ds(off, _CH)], cv)
            ca.wait()
            cb.wait()

            def row(r, c2):
                for kk in range(_H // _LAN):
                    s = pl.ds(kk * _LAN, _LAN)
                    cv[r, s] = jnp.maximum(av[r, s] + bv[r, s] + cv[r, s], 0.0)
                return c2

            lax.fori_loop(0, _CH, row, 0)
            pltpu.sync_copy(cv, S_sh.at[idv], add=True)
            return carry

        lax.fori_loop(0, n_chunks, chunk, 0)
        plsc.subcore_barrier()
        pltpu.sync_copy(S_sh.at[pl.ds(r0, rows_pt)],
                        out_h.at[cid, pl.ds(r0, rows_pt)])
        if tail:
            @pl.when(sid == _NS - 1)
            def _out_tail():
                pltpu.sync_copy(S_sh.at[pl.ds(rows_pt * _NS, tail)],
                                out_h.at[cid, pl.ds(rows_pt * _NS, tail)])

    return k(A, B, C_all, src, dst, zeros)


def _deg_pass(dst, zeros16, N):
    E = dst.shape[0]
    per_w = E // (_NC * _NS)
    n_chunks = per_w // _CH
    rows_pt = (N // _NS) // 8 * 8
    tail = N - rows_pt * _NS
    mesh = plsc.VectorSubcoreMesh(core_axis_name="c", subcore_axis_name="s",
                                  num_cores=_NC, num_subcores=_NS)

    @functools.partial(
        pl.kernel,
        out_type=jax.ShapeDtypeStruct((_NC, N, _LAN), jnp.float32),
        mesh=mesh,
        scratch_types=[
            pltpu.VMEM((_CH,), jnp.int32),
            pltpu.VMEM((_CH, _LAN), jnp.float32),
            pltpu.VMEM_SHARED((N, _LAN), jnp.float32),
        ],
    )
    def k(dst_h, z_h, out_h, idv, ones_v, S_sh):
        cid = lax.axis_index("c")
        sid = lax.axis_index("s")
        r0 = sid * rows_pt
        pltpu.sync_copy(z_h.at[pl.ds(r0, rows_pt)], S_sh.at[pl.ds(r0, rows_pt)])
        if tail:
            @pl.when(sid == _NS - 1)
            def _zero_tail():
                pltpu.sync_copy(z_h.at[pl.ds(rows_pt * _NS, tail)],
                                S_sh.at[pl.ds(rows_pt * _NS, tail)])

        def fill(r, c2):
            ones_v[r, pl.ds(0, _LAN)] = jnp.full((_LAN,), 1.0, jnp.float32)
            return c2

        lax.fori_loop(0, _CH, fill, 0)
        plsc.subcore_barrier()
        base = (cid * _NS + sid) * per_w

        def chunk(i, carry):
            off = base + i * _CH
            pltpu.sync_copy(dst_h.at[pl.ds(off, _CH)], idv)
            pltpu.sync_copy(ones_v, S_sh.at[idv], add=True)
            return carry

        lax.fori_loop(0, n_chunks, chunk, 0)
        plsc.subcore_barrier()
        pltpu.sync_copy(S_sh.at[pl.ds(r0, rows_pt)],
                        out_h.at[cid, pl.ds(r0, rows_pt)])
        if tail:
            @pl.when(sid == _NS - 1)
            def _out_tail():
                pltpu.sync_copy(S_sh.at[pl.ds(rows_pt * _NS, tail)],
                                out_h.at[cid, pl.ds(rows_pt * _NS, tail)])

    return k(dst, zeros16)


def kernel(x, coords, edge_attr, bc_disp, bc_rot, params, edge_index):
    N = x.shape[0]
    src = edge_index[0].astype(jnp.int32)
    dst = edge_index[1].astype(jnp.int32)
    x6 = x[:, 3:]
    c2d = coords[:, 0:3:2]

    (P1, q1), (P2, q2) = params['node_enc']
    (U1, c1), (U2, c2) = params['edge_enc']
    mp = params['mp']
    Ws = jnp.stack([p['msg'][0][0][0:_H] for p in mp])
    Wd = jnp.stack([p['msg'][0][0][_H:2 * _H] for p in mp])
    We = jnp.stack([p['msg'][0][0][2 * _H:3 * _H] for p in mp])
    b1 = jnp.stack([p['msg'][0][1].reshape(1, _H) for p in mp])
    W2 = jnp.stack([p['msg'][1][0] for p in mp])
    b2 = jnp.stack([p['msg'][1][1].reshape(1, _H) for p in mp])
    V1h = jnp.stack([p['node'][0][0][0:_H] for p in mp])
    V1a = jnp.stack([p['node'][0][0][_H:2 * _H] for p in mp])
    d1 = jnp.stack([p['node'][0][1].reshape(1, _H) for p in mp])
    V2 = jnp.stack([p['node'][1][0] for p in mp])
    d2 = jnp.stack([p['node'][1][1].reshape(1, _H) for p in mp])
    (D1, e1), (D2, e2), (D3, e3) = params['dec']
    D1h, D1c = D1[0:_H], D1[_H:_H + 2]
    e1, e2, e3 = e1.reshape(1, -1), e2.reshape(1, -1), e3.reshape(1, -1)

    zeros = jnp.zeros((N, _H), jnp.float32)
    zeros16 = jnp.zeros((N, _LAN), jnp.float32)

    Wcomb, ccomb = _wprep(U2, c2.reshape(1, _H), We)
    C_all = _edge_enc(edge_attr, U1, c1.reshape(1, _H), Wcomb, ccomb)
    h = _node_enc(x6, P1, q1.reshape(1, _H), P2, q2.reshape(1, _H))
    deg = _deg_pass(dst, zeros16, N)

    for l in range(_NLAYERS):
        A, B = _ab(h, Ws[l], Wd[l], b1[l])
        S2 = _edge_pass(A, B, C_all, src, dst, zeros, l)
        h = _node_upd(S2, deg, h, W2[l], b2[l], V1h[l], V1a[l], d1[l],
                      V2[l], d2[l])

    return _dec(h, c2d, bc_disp, bc_rot, D1h, D1c, e1, D2, e2, D3, e3)
